# baseline (speedup 1.0000x reference)
import functools

import jax
import jax.numpy as jnp
from jax import lax
from jax.experimental import pallas as pl
from jax.experimental.pallas import tpu as pltpu
from jax.experimental.pallas import tpu_sc as plsc

N_NODES = 4096
K_KNN = 15
IN_CH = 256
HID = 128


def _mm_body(a_ref, b_ref, o_ref, acc_ref, *, nk, trans_b):
    @pl.when(pl.program_id(2) == 0)
    def _():
        acc_ref[...] = jnp.zeros_like(acc_ref)

    a = a_ref[...]
    b = b_ref[...]
    if trans_b:
        acc_ref[...] += jax.lax.dot_general(
            a, b, (((1,), (1,)), ((), ())), preferred_element_type=jnp.float32)
    else:
        acc_ref[...] += jnp.dot(a, b, preferred_element_type=jnp.float32)

    @pl.when(pl.program_id(2) == nk - 1)
    def _():
        o_ref[...] = acc_ref[...]


def _matmul(a, b, trans_b=False, bm=1024, bn=1024, bk=512):
    m, ka = a.shape
    if trans_b:
        n, kb = b.shape
    else:
        kb, n = b.shape
    nk = ka // bk
    grid = (m // bm, n // bn, nk)
    if trans_b:
        b_spec = pl.BlockSpec((bn, bk), lambda i, j, k: (j, k))
    else:
        b_spec = pl.BlockSpec((bk, bn), lambda i, j, k: (k, j))
    return pl.pallas_call(
        functools.partial(_mm_body, nk=nk, trans_b=trans_b),
        grid=grid,
        in_specs=[pl.BlockSpec((bm, bk), lambda i, j, k: (i, k)), b_spec],
        out_specs=pl.BlockSpec((bm, bn), lambda i, j, k: (i, j)),
        out_shape=jax.ShapeDtypeStruct((m, n), jnp.float32),
        scratch_shapes=[pltpu.VMEM((bm, bn), jnp.float32)],
    )(a, b)


def _knn_body(xb_ref, xall_ref, idx_ref, d_ref, *, bi, n, k):
    i = pl.program_id(0)
    xb = xb_ref[...]
    xall = xall_ref[...]
    sqb = jnp.sum(xb * xb, axis=1, keepdims=True)
    sqall = jnp.sum(xall * xall, axis=1)[None, :]
    prod = jax.lax.dot_general(
        xb, xall, (((1,), (1,)), ((), ())), preferred_element_type=jnp.float32)
    d = sqb + sqall - 2.0 * prod
    col = jax.lax.broadcasted_iota(jnp.int32, (bi, n), 1)
    grow = i * bi + jax.lax.broadcasted_iota(jnp.int32, (bi, n), 0)
    inf = jnp.float32(jnp.inf)
    d = jnp.where(col == grow, inf, d)
    d_ref[...] = d
    for j in range(k):
        m = jnp.min(d_ref[...], axis=1, keepdims=True)
        hit = d_ref[...] <= m
        idx = jnp.min(jnp.where(hit, col, n), axis=1)
        idx_ref[:, j] = idx
        d_ref[...] = jnp.where(col == idx[:, None], inf, d_ref[...])
    idx_ref[:, k] = idx_ref[:, 0]


def _knn_graph(x, k):
    n = x.shape[0]
    bi = 256
    idx_pad = pl.pallas_call(
        functools.partial(_knn_body, bi=bi, n=n, k=k),
        grid=(n // bi,),
        in_specs=[
            pl.BlockSpec((bi, IN_CH), lambda i: (i, 0)),
            pl.BlockSpec((n, IN_CH), lambda i: (0, 0)),
        ],
        out_specs=pl.BlockSpec((bi, 128), lambda i: (i, 0)),
        out_shape=jax.ShapeDtypeStruct((n, 128), jnp.int32),
        scratch_shapes=[pltpu.VMEM((bi, n), jnp.float32)],
    )(x, x)
    return idx_pad


_NW = 32
_NODES_PER_W = N_NODES // _NW
_NCHUNK = 8
_IDX_PER_CHUNK = _NCHUNK * 16


def _scmax_kernel(x_hbm, idx_hbm, out_hbm, idx_v, g_v, out_v, sem):
    wid = lax.axis_index("s") * 2 + lax.axis_index("c")
    nvec = IN_CH // 16

    def chunk_body(j, _):
        node0 = wid * _NODES_PER_W + j * _NCHUNK
        pltpu.sync_copy(idx_hbm.at[pl.ds(node0 * 16, _IDX_PER_CHUNK)], idx_v)
        pltpu.async_copy(x_hbm.at[idx_v], g_v, sem).wait()

        def node_body(n, _):
            def nb_body(nb, accs):
                r = n * 16 + nb
                return [jnp.maximum(a, g_v[r, pl.ds(c * 16, 16)])
                        for c, a in enumerate(accs)]
            accs = [g_v[n * 16, pl.ds(c * 16, 16)] for c in range(nvec)]
            accs = lax.fori_loop(1, 16, nb_body, accs)
            for c in range(nvec):
                out_v[n, pl.ds(c * 16, 16)] = accs[c]
            return 0

        lax.fori_loop(0, _NCHUNK, node_body, 0)
        pltpu.sync_copy(out_v, out_hbm.at[pl.ds(node0, _NCHUNK)])
        return 0

    lax.fori_loop(0, _NODES_PER_W // _NCHUNK, chunk_body, 0)


def _sc_knn_max(x, idx16_flat):
    mesh = plsc.VectorSubcoreMesh(core_axis_name="c", subcore_axis_name="s")
    f = pl.kernel(
        _scmax_kernel,
        mesh=mesh,
        out_type=jax.ShapeDtypeStruct((N_NODES, IN_CH), jnp.float32),
        scratch_types=[
            pltpu.VMEM((_IDX_PER_CHUNK,), jnp.int32),
            pltpu.VMEM((_IDX_PER_CHUNK, IN_CH), jnp.float32),
            pltpu.VMEM((_NCHUNK, IN_CH), jnp.float32),
            pltpu.SemaphoreType.DMA,
        ],
    )
    return f(x, idx16_flat)


def _smax_body(q_ref, kt_ref, a_ref, s_ref, *, bi, n):
    qk = jax.lax.dot_general(
        q_ref[...], kt_ref[...], (((1,), (1,)), ((), ())),
        preferred_element_type=jnp.float32,
        precision=jax.lax.Precision.HIGHEST)
    a = a_ref[...]
    mask = a > 0.0
    neg_inf = jnp.float32(-jnp.inf)
    mx = jnp.max(jnp.where(mask, qk, neg_inf), axis=1, keepdims=True)
    mx = jnp.where(jnp.isfinite(mx), mx, 0.0)
    p = jnp.where(mask, jnp.exp(qk - mx), 0.0) * a
    s = jnp.sum(p, axis=1, keepdims=True)
    s_ref[...] = p / (s + 1e-16)


def _masked_softmax(q, k, a):
    n = a.shape[0]
    bi = 512
    return pl.pallas_call(
        functools.partial(_smax_body, bi=bi, n=n),
        grid=(n // bi,),
        in_specs=[
            pl.BlockSpec((bi, HID), lambda i: (i, 0)),
            pl.BlockSpec((n, HID), lambda i: (0, 0)),
            pl.BlockSpec((bi, n), lambda i: (i, 0)),
        ],
        out_specs=pl.BlockSpec((bi, n), lambda i: (i, 0)),
        out_shape=jax.ShapeDtypeStruct((n, n), jnp.float32),
    )(q, k, a)


def kernel(x, edge_index, W_dev, W_q, W_k):
    row = edge_index[0].astype(jnp.int32)
    col = edge_index[1].astype(jnp.int32)

    idx_pad = _knn_graph(x, K_KNN)
    idx16 = idx_pad[:, :16].reshape(-1)

    m_knn = _sc_knn_max(x, idx16)
    m_in = jax.ops.segment_max(x[col], row, num_segments=N_NODES)
    m = jnp.maximum(m_knn, m_in)
    agg = m - x
    agg = jnp.where(jnp.isfinite(agg), agg, 0.0)
    features = agg @ W_dev

    q = features @ W_q
    k = features @ W_k

    A = jnp.zeros((N_NODES, N_NODES), jnp.float32).at[row, col].add(1.0)
    S = _masked_softmax(q, k, A)

    T = _matmul(S, A)
    A_s = _matmul(T, S, trans_b=True)
    return A_s


# speedup vs baseline: 1.0904x; 1.0904x over previous
import functools

import jax
import jax.numpy as jnp
from jax import lax
from jax.experimental import pallas as pl
from jax.experimental.pallas import tpu as pltpu
from jax.experimental.pallas import tpu_sc as plsc

N_NODES = 4096
K_KNN = 15
IN_CH = 256
HID = 128


def _mm_body(a_ref, b_ref, o_ref, acc_ref, *, nk, trans_b):
    @pl.when(pl.program_id(2) == 0)
    def _():
        acc_ref[...] = jnp.zeros_like(acc_ref)

    a = a_ref[...]
    b = b_ref[...]
    if trans_b:
        acc_ref[...] += jax.lax.dot_general(
            a, b, (((1,), (1,)), ((), ())), preferred_element_type=jnp.float32)
    else:
        acc_ref[...] += jnp.dot(a, b, preferred_element_type=jnp.float32)

    @pl.when(pl.program_id(2) == nk - 1)
    def _():
        o_ref[...] = acc_ref[...]


def _matmul(a, b, trans_b=False, bm=2048, bn=1024, bk=512):
    m, ka = a.shape
    if trans_b:
        n, kb = b.shape
    else:
        kb, n = b.shape
    nk = ka // bk
    grid = (m // bm, n // bn, nk)
    if trans_b:
        b_spec = pl.BlockSpec((bn, bk), lambda i, j, k: (j, k))
    else:
        b_spec = pl.BlockSpec((bk, bn), lambda i, j, k: (k, j))
    return pl.pallas_call(
        functools.partial(_mm_body, nk=nk, trans_b=trans_b),
        grid=grid,
        in_specs=[pl.BlockSpec((bm, bk), lambda i, j, k: (i, k)), b_spec],
        out_specs=pl.BlockSpec((bm, bn), lambda i, j, k: (i, j)),
        out_shape=jax.ShapeDtypeStruct((m, n), jnp.float32),
        scratch_shapes=[pltpu.VMEM((bm, bn), jnp.float32)],
    )(a, b)


def _knn_body(xb_ref, xall_ref, idx_ref, d_ref, *, bi, n, k):
    i = pl.program_id(0)
    xb = xb_ref[...]
    xall = xall_ref[...]
    sqb = jnp.sum(xb * xb, axis=1, keepdims=True)
    sqall = jnp.sum(xall * xall, axis=1)[None, :]
    prod = jax.lax.dot_general(
        xb, xall, (((1,), (1,)), ((), ())), preferred_element_type=jnp.float32)
    d = sqb + sqall - 2.0 * prod
    col = jax.lax.broadcasted_iota(jnp.int32, (bi, n), 1)
    grow = i * bi + jax.lax.broadcasted_iota(jnp.int32, (bi, n), 0)
    inf = jnp.float32(jnp.inf)
    d = jnp.where(col == grow, inf, d)
    d_ref[...] = d
    for j in range(k):
        m = jnp.min(d_ref[...], axis=1, keepdims=True)
        hit = d_ref[...] <= m
        idx = jnp.min(jnp.where(hit, col, n), axis=1)
        idx_ref[:, j] = idx
        d_ref[...] = jnp.where(col == idx[:, None], inf, d_ref[...])
    idx_ref[:, k] = idx_ref[:, 0]


def _knn_graph(x, k):
    n = x.shape[0]
    bi = 256
    idx_pad = pl.pallas_call(
        functools.partial(_knn_body, bi=bi, n=n, k=k),
        grid=(n // bi,),
        in_specs=[
            pl.BlockSpec((bi, IN_CH), lambda i: (i, 0)),
            pl.BlockSpec((n, IN_CH), lambda i: (0, 0)),
        ],
        out_specs=pl.BlockSpec((bi, 128), lambda i: (i, 0)),
        out_shape=jax.ShapeDtypeStruct((n, 128), jnp.int32),
        scratch_shapes=[pltpu.VMEM((bi, n), jnp.float32)],
    )(x, x)
    return idx_pad


_NW = 32
_NODES_PER_W = N_NODES // _NW
_NCHUNK = 8
_IDX_PER_CHUNK = _NCHUNK * 16


def _scmax_kernel(x_hbm, idx_hbm, out_hbm, idx_v, g_v, out_v, sem):
    wid = lax.axis_index("s") * 2 + lax.axis_index("c")
    nvec = IN_CH // 16

    def chunk_body(j, _):
        node0 = wid * _NODES_PER_W + j * _NCHUNK
        pltpu.sync_copy(idx_hbm.at[pl.ds(node0 * 16, _IDX_PER_CHUNK)], idx_v)
        pltpu.async_copy(x_hbm.at[idx_v], g_v, sem).wait()

        def node_body(n, _):
            def nb_body(nb, accs):
                r = n * 16 + nb
                return [jnp.maximum(a, g_v[r, pl.ds(c * 16, 16)])
                        for c, a in enumerate(accs)]
            accs = [g_v[n * 16, pl.ds(c * 16, 16)] for c in range(nvec)]
            accs = lax.fori_loop(1, 16, nb_body, accs)
            for c in range(nvec):
                out_v[n, pl.ds(c * 16, 16)] = accs[c]
            return 0

        lax.fori_loop(0, _NCHUNK, node_body, 0)
        pltpu.sync_copy(out_v, out_hbm.at[pl.ds(node0, _NCHUNK)])
        return 0

    lax.fori_loop(0, _NODES_PER_W // _NCHUNK, chunk_body, 0)


def _sc_knn_max(x, idx16_flat):
    mesh = plsc.VectorSubcoreMesh(core_axis_name="c", subcore_axis_name="s")
    f = pl.kernel(
        _scmax_kernel,
        mesh=mesh,
        out_type=jax.ShapeDtypeStruct((N_NODES, IN_CH), jnp.float32),
        scratch_types=[
            pltpu.VMEM((_IDX_PER_CHUNK,), jnp.int32),
            pltpu.VMEM((_IDX_PER_CHUNK, IN_CH), jnp.float32),
            pltpu.VMEM((_NCHUNK, IN_CH), jnp.float32),
            pltpu.SemaphoreType.DMA,
        ],
    )
    return f(x, idx16_flat)


def _smax_body(q_ref, kt_ref, a_ref, s_ref, *, bi, n):
    qk = jax.lax.dot_general(
        q_ref[...], kt_ref[...], (((1,), (1,)), ((), ())),
        preferred_element_type=jnp.float32,
        precision=jax.lax.Precision.HIGHEST)
    a = a_ref[...]
    mask = a > 0.0
    neg_inf = jnp.float32(-jnp.inf)
    mx = jnp.max(jnp.where(mask, qk, neg_inf), axis=1, keepdims=True)
    mx = jnp.where(jnp.isfinite(mx), mx, 0.0)
    p = jnp.where(mask, jnp.exp(qk - mx), 0.0) * a
    s = jnp.sum(p, axis=1, keepdims=True)
    s_ref[...] = p / (s + 1e-16)


def _masked_softmax(q, k, a):
    n = a.shape[0]
    bi = 512
    return pl.pallas_call(
        functools.partial(_smax_body, bi=bi, n=n),
        grid=(n // bi,),
        in_specs=[
            pl.BlockSpec((bi, HID), lambda i: (i, 0)),
            pl.BlockSpec((n, HID), lambda i: (0, 0)),
            pl.BlockSpec((bi, n), lambda i: (i, 0)),
        ],
        out_specs=pl.BlockSpec((bi, n), lambda i: (i, 0)),
        out_shape=jax.ShapeDtypeStruct((n, n), jnp.float32),
    )(q, k, a)


def kernel(x, edge_index, W_dev, W_q, W_k):
    row = edge_index[0].astype(jnp.int32)
    col = edge_index[1].astype(jnp.int32)

    idx_pad = _knn_graph(x, K_KNN)
    idx16 = idx_pad[:, :16].reshape(-1)

    m_knn = _sc_knn_max(x, idx16)
    m_in = jax.ops.segment_max(x[col], row, num_segments=N_NODES)
    m = jnp.maximum(m_knn, m_in)
    agg = m - x
    agg = jnp.where(jnp.isfinite(agg), agg, 0.0)
    features = agg @ W_dev

    q = features @ W_q
    k = features @ W_k

    A = jnp.zeros((N_NODES, N_NODES), jnp.float32).at[row, col].add(1.0)
    S = _masked_softmax(q, k, A)

    T = _matmul(S, A)
    A_s = _matmul(T, S, trans_b=True)
    return A_s


# speedup vs baseline: 1.1177x; 1.0250x over previous
import functools

import jax
import jax.numpy as jnp
from jax import lax
from jax.experimental import pallas as pl
from jax.experimental.pallas import tpu as pltpu
from jax.experimental.pallas import tpu_sc as plsc

N_NODES = 4096
K_KNN = 15
IN_CH = 256
HID = 128


def _mm_body(a_ref, b_ref, o_ref, acc_ref, *, nk, trans_b):
    @pl.when(pl.program_id(2) == 0)
    def _():
        acc_ref[...] = jnp.zeros_like(acc_ref)

    a = a_ref[...]
    b = b_ref[...]
    if trans_b:
        acc_ref[...] += jax.lax.dot_general(
            a, b, (((1,), (1,)), ((), ())), preferred_element_type=jnp.float32)
    else:
        acc_ref[...] += jnp.dot(a, b, preferred_element_type=jnp.float32)

    @pl.when(pl.program_id(2) == nk - 1)
    def _():
        o_ref[...] = acc_ref[...]


def _matmul(a, b, trans_b=False, bm=2048, bn=1024, bk=1024):
    m, ka = a.shape
    if trans_b:
        n, kb = b.shape
    else:
        kb, n = b.shape
    nk = ka // bk
    grid = (m // bm, n // bn, nk)
    if trans_b:
        b_spec = pl.BlockSpec((bn, bk), lambda i, j, k: (j, k))
    else:
        b_spec = pl.BlockSpec((bk, bn), lambda i, j, k: (k, j))
    return pl.pallas_call(
        functools.partial(_mm_body, nk=nk, trans_b=trans_b),
        grid=grid,
        in_specs=[pl.BlockSpec((bm, bk), lambda i, j, k: (i, k)), b_spec],
        out_specs=pl.BlockSpec((bm, bn), lambda i, j, k: (i, j)),
        out_shape=jax.ShapeDtypeStruct((m, n), jnp.float32),
        scratch_shapes=[pltpu.VMEM((bm, bn), jnp.float32)],
    )(a, b)


def _knn_body(xb_ref, xall_ref, idx_ref, d_ref, *, bi, n, k):
    i = pl.program_id(0)
    xb = xb_ref[...]
    xall = xall_ref[...]
    sqb = jnp.sum(xb * xb, axis=1, keepdims=True)
    sqall = jnp.sum(xall * xall, axis=1)[None, :]
    prod = jax.lax.dot_general(
        xb, xall, (((1,), (1,)), ((), ())), preferred_element_type=jnp.float32)
    d = sqb + sqall - 2.0 * prod
    col = jax.lax.broadcasted_iota(jnp.int32, (bi, n), 1)
    grow = i * bi + jax.lax.broadcasted_iota(jnp.int32, (bi, n), 0)
    inf = jnp.float32(jnp.inf)
    d = jnp.where(col == grow, inf, d)
    d_ref[...] = d
    for j in range(k):
        m = jnp.min(d_ref[...], axis=1, keepdims=True)
        hit = d_ref[...] <= m
        idx = jnp.min(jnp.where(hit, col, n), axis=1)
        idx_ref[:, j] = idx
        d_ref[...] = jnp.where(col == idx[:, None], inf, d_ref[...])
    idx_ref[:, k] = idx_ref[:, 0]


def _knn_graph(x, k):
    n = x.shape[0]
    bi = 512
    idx_pad = pl.pallas_call(
        functools.partial(_knn_body, bi=bi, n=n, k=k),
        grid=(n // bi,),
        in_specs=[
            pl.BlockSpec((bi, IN_CH), lambda i: (i, 0)),
            pl.BlockSpec((n, IN_CH), lambda i: (0, 0)),
        ],
        out_specs=pl.BlockSpec((bi, 128), lambda i: (i, 0)),
        out_shape=jax.ShapeDtypeStruct((n, 128), jnp.int32),
        scratch_shapes=[pltpu.VMEM((bi, n), jnp.float32)],
    )(x, x)
    return idx_pad


_NW = 32
_NODES_PER_W = N_NODES // _NW
_NCHUNK = 8
_IDX_PER_CHUNK = _NCHUNK * 16


def _scmax_kernel(x_hbm, idx_hbm, out_hbm, idx_v, g_v, out_v, sem):
    wid = lax.axis_index("s") * 2 + lax.axis_index("c")
    nvec = IN_CH // 16

    def chunk_body(j, _):
        node0 = wid * _NODES_PER_W + j * _NCHUNK
        pltpu.sync_copy(idx_hbm.at[pl.ds(node0 * 16, _IDX_PER_CHUNK)], idx_v)
        pltpu.async_copy(x_hbm.at[idx_v], g_v, sem).wait()

        def node_body(n, _):
            def nb_body(nb, accs):
                r = n * 16 + nb
                return [jnp.maximum(a, g_v[r, pl.ds(c * 16, 16)])
                        for c, a in enumerate(accs)]
            accs = [g_v[n * 16, pl.ds(c * 16, 16)] for c in range(nvec)]
            accs = lax.fori_loop(1, 16, nb_body, accs)
            for c in range(nvec):
                out_v[n, pl.ds(c * 16, 16)] = accs[c]
            return 0

        lax.fori_loop(0, _NCHUNK, node_body, 0)
        pltpu.sync_copy(out_v, out_hbm.at[pl.ds(node0, _NCHUNK)])
        return 0

    lax.fori_loop(0, _NODES_PER_W // _NCHUNK, chunk_body, 0)


def _sc_knn_max(x, idx16_flat):
    mesh = plsc.VectorSubcoreMesh(core_axis_name="c", subcore_axis_name="s")
    f = pl.kernel(
        _scmax_kernel,
        mesh=mesh,
        out_type=jax.ShapeDtypeStruct((N_NODES, IN_CH), jnp.float32),
        scratch_types=[
            pltpu.VMEM((_IDX_PER_CHUNK,), jnp.int32),
            pltpu.VMEM((_IDX_PER_CHUNK, IN_CH), jnp.float32),
            pltpu.VMEM((_NCHUNK, IN_CH), jnp.float32),
            pltpu.SemaphoreType.DMA,
        ],
    )
    return f(x, idx16_flat)


def _smax_body(q_ref, kt_ref, a_ref, s_ref, *, bi, n):
    qk = jax.lax.dot_general(
        q_ref[...], kt_ref[...], (((1,), (1,)), ((), ())),
        preferred_element_type=jnp.float32,
        precision=jax.lax.Precision.HIGHEST)
    a = a_ref[...]
    mask = a > 0.0
    neg_inf = jnp.float32(-jnp.inf)
    mx = jnp.max(jnp.where(mask, qk, neg_inf), axis=1, keepdims=True)
    mx = jnp.where(jnp.isfinite(mx), mx, 0.0)
    p = jnp.where(mask, jnp.exp(qk - mx), 0.0) * a
    s = jnp.sum(p, axis=1, keepdims=True)
    s_ref[...] = p / (s + 1e-16)


def _masked_softmax(q, k, a):
    n = a.shape[0]
    bi = 512
    return pl.pallas_call(
        functools.partial(_smax_body, bi=bi, n=n),
        grid=(n // bi,),
        in_specs=[
            pl.BlockSpec((bi, HID), lambda i: (i, 0)),
            pl.BlockSpec((n, HID), lambda i: (0, 0)),
            pl.BlockSpec((bi, n), lambda i: (i, 0)),
        ],
        out_specs=pl.BlockSpec((bi, n), lambda i: (i, 0)),
        out_shape=jax.ShapeDtypeStruct((n, n), jnp.float32),
    )(q, k, a)


def kernel(x, edge_index, W_dev, W_q, W_k):
    row = edge_index[0].astype(jnp.int32)
    col = edge_index[1].astype(jnp.int32)

    idx_pad = _knn_graph(x, K_KNN)
    idx16 = idx_pad[:, :16].reshape(-1)

    m_knn = _sc_knn_max(x, idx16)
    m_in = jax.ops.segment_max(x[col], row, num_segments=N_NODES)
    m = jnp.maximum(m_knn, m_in)
    agg = m - x
    agg = jnp.where(jnp.isfinite(agg), agg, 0.0)
    features = agg @ W_dev

    q = features @ W_q
    k = features @ W_k

    A = jnp.zeros((N_NODES, N_NODES), jnp.float32).at[row, col].add(1.0)
    S = _masked_softmax(q, k, A)

    T = _matmul(S, A)
    A_s = _matmul(T, S, trans_b=True)
    return A_s
